# Initial kernel scaffold; baseline (speedup 1.0000x reference)
import jax
import jax.numpy as jnp
from jax.experimental import pallas as pl


def kernel(x, edge_index, Wl1, bl1, Wr1, g1, beta1, W1, bW1, W2, bW2, Wl2, bl2, Wr2, g2, beta2, W3, bW3, W4, bW4):
    raise NotImplementedError("write your pallas kernel here")


# speedup vs baseline: 4.9468x; 4.9468x over previous
import functools

import jax
import jax.numpy as jnp
from jax import lax
from jax.experimental import pallas as pl
from jax.experimental.pallas import tpu as pltpu
from jax.experimental.pallas import tpu_sc as plsc

N_NODES = 10000
N_EDGES = 320000
NC = 2
NS = 16
NW = NC * NS
EB = 128
NB_TOTAL = N_EDGES // EB
EPS = 1e-5

RPT = 624

_SC_MESH = dict(core_axis_name="c", subcore_axis_name="s",
                num_cores=NC, num_subcores=NS)


def _sc_conv1_body(x_hbm, src_hbm, dst_hbm, zrow_hbm,
                   agg_out,
                   agg_sh, sidx, didx, rows, sem):
    c = lax.axis_index("c")
    s = lax.axis_index("s")
    w = c * NS + s
    r0 = s * RPT

    pltpu.sync_copy(zrow_hbm.at[pl.ds(0, RPT)], agg_sh.at[pl.ds(r0, RPT)])

    @pl.when(s == NS - 1)
    def _():
        pltpu.sync_copy(zrow_hbm.at[pl.ds(0, 16)],
                        agg_sh.at[pl.ds(NS * RPT, 16)])

    plsc.subcore_barrier()

    nb = 78 + (w < 4).astype(jnp.int32)
    g0 = 78 * w + jnp.minimum(w, 4)

    def body(i, carry):
        e0 = (g0 + i) * EB
        pltpu.sync_copy(src_hbm.at[pl.ds(e0, EB)], sidx)
        pltpu.sync_copy(dst_hbm.at[pl.ds(e0, EB)], didx)
        pltpu.async_copy(x_hbm.at[sidx], rows, sem).wait()
        pltpu.sync_copy(rows, agg_sh.at[didx], add=True)
        return carry

    lax.fori_loop(0, nb, body, 0)
    plsc.subcore_barrier()

    pltpu.sync_copy(agg_sh.at[pl.ds(r0, RPT)], agg_out.at[c, pl.ds(r0, RPT)])

    @pl.when(s == NS - 1)
    def _():
        pltpu.sync_copy(agg_sh.at[pl.ds(NS * RPT, 16)],
                        agg_out.at[c, pl.ds(NS * RPT, 16)])


def _sc_count_body(dst_hbm, zrow_hbm, ones_hbm,
                   cnt_out,
                   cnt_sh, didx, ones, sem):
    c = lax.axis_index("c")
    s = lax.axis_index("s")
    w = c * NS + s
    r0 = s * RPT

    pltpu.sync_copy(zrow_hbm.at[pl.ds(0, RPT)], cnt_sh.at[pl.ds(r0, RPT)])

    @pl.when(s == NS - 1)
    def _():
        pltpu.sync_copy(zrow_hbm.at[pl.ds(0, 16)],
                        cnt_sh.at[pl.ds(NS * RPT, 16)])

    pltpu.sync_copy(ones_hbm.at[pl.ds(0, EB)], ones)
    plsc.subcore_barrier()

    nb = 78 + (w < 4).astype(jnp.int32)
    g0 = 78 * w + jnp.minimum(w, 4)

    def body(i, carry):
        e0 = (g0 + i) * EB
        pltpu.sync_copy(dst_hbm.at[pl.ds(e0, EB)], didx)
        pltpu.sync_copy(ones, cnt_sh.at[didx], add=True)
        return carry

    lax.fori_loop(0, nb, body, 0)
    plsc.subcore_barrier()

    pltpu.sync_copy(cnt_sh.at[pl.ds(r0, RPT)], cnt_out.at[c, pl.ds(r0, RPT)])

    @pl.when(s == NS - 1)
    def _():
        pltpu.sync_copy(cnt_sh.at[pl.ds(NS * RPT, 16)],
                        cnt_out.at[c, pl.ds(NS * RPT, 16)])


def _sc_conv2_body(h_hbm, src_hbm, dst_hbm, zrow_hbm,
                   agg_out,
                   agg_sh, sidx, didx, rows, sem):
    c = lax.axis_index("c")
    s = lax.axis_index("s")
    r0 = s * RPT

    pltpu.sync_copy(zrow_hbm.at[pl.ds(0, RPT)], agg_sh.at[pl.ds(r0, RPT)])

    @pl.when(s == NS - 1)
    def _():
        pltpu.sync_copy(zrow_hbm.at[pl.ds(0, 16)],
                        agg_sh.at[pl.ds(NS * RPT, 16)])

    plsc.subcore_barrier()

    base16 = jnp.full((16,), 0, jnp.int32) + c * N_NODES
    nb = 156 + (s < 4).astype(jnp.int32)
    g0 = 156 * s + jnp.minimum(s, 4)

    def body(i, carry):
        e0 = (g0 + i) * EB
        pltpu.sync_copy(src_hbm.at[pl.ds(e0, EB)], sidx)
        pltpu.sync_copy(dst_hbm.at[pl.ds(e0, EB)], didx)
        for k in range(EB // 16):
            sl = pl.ds(k * 16, 16)
            sidx[sl] = sidx[sl] + base16
        pltpu.async_copy(h_hbm.at[sidx], rows, sem).wait()
        pltpu.sync_copy(rows, agg_sh.at[didx], add=True)
        return carry

    lax.fori_loop(0, nb, body, 0)
    plsc.subcore_barrier()

    pltpu.sync_copy(agg_sh.at[pl.ds(r0, RPT)], agg_out.at[c, pl.ds(r0, RPT)])

    @pl.when(s == NS - 1)
    def _():
        pltpu.sync_copy(agg_sh.at[pl.ds(NS * RPT, 16)],
                        agg_out.at[c, pl.ds(NS * RPT, 16)])


_DN = (((1,), (1,)), ((), ()))


def _dot(a, b):
    return lax.dot_general(a, b, _DN, preferred_element_type=jnp.float32)


def _tc1_body(x_ref, p_ref, cntt_ref, wl1_ref, bl1_ref, wr1_ref, g1_ref,
              b1_ref, w1_ref, bw1_ref, w2_ref, bw2_ref, hs_ref):
    cnt = cntt_ref[0][:, 0:1] + cntt_ref[1][:, 0:1]
    mean = (p_ref[0] + p_ref[1]) / jnp.clip(cnt, 1.0)
    pre = _dot(mean, wl1_ref[...]) + bl1_ref[...] + _dot(x_ref[...], wr1_ref[...])
    mu = jnp.mean(pre, axis=-1, keepdims=True)
    var = jnp.mean((pre - mu) ** 2, axis=-1, keepdims=True)
    h = jnp.maximum((pre - mu) * lax.rsqrt(var + EPS) * g1_ref[...] + b1_ref[...], 0.0)
    t = jnp.maximum(_dot(h, w1_ref[...]) + bw1_ref[...], 0.0)
    h2 = jnp.maximum(_dot(t, w2_ref[...]) + bw2_ref[...], 0.0)
    hs_ref[0] = h2[:, :128]
    hs_ref[1] = h2[:, 128:]


def _tc2_body(hs_ref, q_ref, cntt_ref, wl2_ref, bl2_ref, wr2_ref, g2_ref,
              b2_ref, w3_ref, bw3_ref, w4_ref, bw4_ref, o_ref):
    cnt = cntt_ref[0][:, 0:1] + cntt_ref[1][:, 0:1]
    icnt = 1.0 / jnp.clip(cnt, 1.0)
    mean = jnp.concatenate([q_ref[0], q_ref[1]], axis=1) * icnt
    hfull = jnp.concatenate([hs_ref[0], hs_ref[1]], axis=1)
    pre = _dot(mean, wl2_ref[...]) + bl2_ref[...] + _dot(hfull, wr2_ref[...])
    mu = jnp.mean(pre, axis=-1, keepdims=True)
    var = jnp.mean((pre - mu) ** 2, axis=-1, keepdims=True)
    h = jnp.maximum((pre - mu) * lax.rsqrt(var + EPS) * g2_ref[...] + b2_ref[...], 0.0)
    t = jnp.maximum(_dot(h, w3_ref[...]) + bw3_ref[...], 0.0)
    o_ref[...] = _dot(t, w4_ref[...]) + bw4_ref[...]


def _full(shape):
    return pl.BlockSpec(shape, lambda i: tuple(0 for _ in shape))


def _sc_conv1(x, src, dst, zrow):
    return pl.kernel(
        _sc_conv1_body,
        out_type=jax.ShapeDtypeStruct((NC, N_NODES, 128), jnp.float32),
        mesh=plsc.VectorSubcoreMesh(**_SC_MESH),
        scratch_types=[
            pltpu.VMEM_SHARED((N_NODES, 128), jnp.float32),
            pltpu.VMEM((EB,), jnp.int32),
            pltpu.VMEM((EB,), jnp.int32),
            pltpu.VMEM((EB, 128), jnp.float32),
            pltpu.SemaphoreType.DMA,
        ],
    )(x, src, dst, zrow)


def _sc_count(dst, zrow, ones_in):
    return pl.kernel(
        _sc_count_body,
        out_type=jax.ShapeDtypeStruct((NC, N_NODES, 128), jnp.float32),
        mesh=plsc.VectorSubcoreMesh(**_SC_MESH),
        scratch_types=[
            pltpu.VMEM_SHARED((N_NODES, 128), jnp.float32),
            pltpu.VMEM((EB,), jnp.int32),
            pltpu.VMEM((EB, 128), jnp.float32),
            pltpu.SemaphoreType.DMA,
        ],
    )(dst, zrow, ones_in)


def _sc_conv2(hflat, src, dst, zrow):
    return pl.kernel(
        _sc_conv2_body,
        out_type=jax.ShapeDtypeStruct((NC, N_NODES, 128), jnp.float32),
        mesh=plsc.VectorSubcoreMesh(**_SC_MESH),
        scratch_types=[
            pltpu.VMEM_SHARED((N_NODES, 128), jnp.float32),
            pltpu.VMEM((EB,), jnp.int32),
            pltpu.VMEM((EB,), jnp.int32),
            pltpu.VMEM((EB, 128), jnp.float32),
            pltpu.SemaphoreType.DMA,
        ],
    )(hflat, src, dst, zrow)


def kernel(x, edge_index, Wl1, bl1, Wr1, g1, beta1, W1, bW1, W2, bW2,
           Wl2, bl2, Wr2, g2, beta2, W3, bW3, W4, bW4):
    ei = edge_index.astype(jnp.int32)
    src, dst = ei[0], ei[1]
    zrow = jnp.zeros((RPT, 128), jnp.float32)
    ones_in = jnp.ones((EB, 128), jnp.float32)

    agg1 = _sc_conv1(x, src, dst, zrow)
    cntt = _sc_count(dst, zrow, ones_in)

    R = 1000
    grid = (N_NODES // R,)
    b1 = bl1.reshape(1, -1)
    gg1 = g1.reshape(1, -1)
    bb1 = beta1.reshape(1, -1)
    hs = pl.pallas_call(
        _tc1_body,
        grid=grid,
        in_specs=[
            pl.BlockSpec((R, 128), lambda i: (i, 0)),
            pl.BlockSpec((NC, R, 128), lambda i: (0, i, 0)),
            pl.BlockSpec((NC, R, 128), lambda i: (0, i, 0)),
            _full(Wl1.shape), _full(b1.shape), _full(Wr1.shape),
            _full(gg1.shape), _full(bb1.shape),
            _full(W1.shape), _full((1, 512)),
            _full(W2.shape), _full((1, 256)),
        ],
        out_specs=pl.BlockSpec((NC, R, 128), lambda i: (0, i, 0)),
        out_shape=jax.ShapeDtypeStruct((NC, N_NODES, 128), jnp.float32),
    )(x, agg1, cntt, Wl1, b1, Wr1, gg1, bb1,
      W1, bW1.reshape(1, -1), W2, bW2.reshape(1, -1))

    hflat = hs.reshape(NC * N_NODES, 128)
    agg2 = _sc_conv2(hflat, src, dst, zrow)

    out = pl.pallas_call(
        _tc2_body,
        grid=grid,
        in_specs=[
            pl.BlockSpec((NC, R, 128), lambda i: (0, i, 0)),
            pl.BlockSpec((NC, R, 128), lambda i: (0, i, 0)),
            pl.BlockSpec((NC, R, 128), lambda i: (0, i, 0)),
            _full(Wl2.shape), _full((1, 128)), _full(Wr2.shape),
            _full((1, 128)), _full((1, 128)),
            _full(W3.shape), _full((1, 256)),
            _full(W4.shape), _full((1, 128)),
        ],
        out_specs=pl.BlockSpec((R, 128), lambda i: (i, 0)),
        out_shape=jax.ShapeDtypeStruct((N_NODES, 128), jnp.float32),
    )(hs, agg2, cntt, Wl2, bl2.reshape(1, -1), Wr2,
      g2.reshape(1, -1), beta2.reshape(1, -1),
      W3, bW3.reshape(1, -1), W4, bW4.reshape(1, -1))

    return out


# speedup vs baseline: 8.8868x; 1.7965x over previous
import functools

import jax
import jax.numpy as jnp
from jax import lax
from jax.experimental import pallas as pl
from jax.experimental.pallas import tpu as pltpu
from jax.experimental.pallas import tpu_sc as plsc

N_NODES = 10000
N_EDGES = 320000
NC = 2
NS = 16
NW = NC * NS
EB = 128
NB_TOTAL = N_EDGES // EB
EPS = 1e-5

RPT = 624

_SC_MESH = dict(core_axis_name="c", subcore_axis_name="s",
                num_cores=NC, num_subcores=NS)


def _zero_slice(zrow_hbm, sh, s):
    r0 = s * RPT
    pltpu.sync_copy(zrow_hbm.at[pl.ds(0, RPT)], sh.at[pl.ds(r0, RPT)])

    @pl.when(s == NS - 1)
    def _():
        pltpu.sync_copy(zrow_hbm.at[pl.ds(0, 16)], sh.at[pl.ds(NS * RPT, 16)])


def _writeback_slice(sh, out, c, s):
    r0 = s * RPT
    pltpu.sync_copy(sh.at[pl.ds(r0, RPT)], out.at[c, pl.ds(r0, RPT)])

    @pl.when(s == NS - 1)
    def _():
        pltpu.sync_copy(sh.at[pl.ds(NS * RPT, 16)],
                        out.at[c, pl.ds(NS * RPT, 16)])


def _sc_conv1_body(x_hbm, src_hbm, dst_hbm, zrow_hbm,
                   agg_out,
                   agg_sh, sidx0, sidx1, didx0, didx1,
                   rows0, rows1, gsem0, gsem1, isem0, isem1):
    c = lax.axis_index("c")
    s = lax.axis_index("s")
    w = c * NS + s

    _zero_slice(zrow_hbm, agg_sh, s)
    plsc.subcore_barrier()

    rows = (rows0, rows1)
    gsem = (gsem0, gsem1)
    sidx = (sidx0, sidx1)
    didx = (didx0, didx1)
    isem = (isem0, isem1)
    g0 = 78 * w

    def fire_idx(j, b):
        e0 = (g0 + j) * EB
        pltpu.async_copy(src_hbm.at[pl.ds(e0, EB)], sidx[b], isem[b])
        pltpu.async_copy(dst_hbm.at[pl.ds(e0, EB)], didx[b], isem[b])

    def wait_idx(b):
        pltpu.make_async_copy(src_hbm.at[pl.ds(0, EB)], sidx[b], isem[b]).wait()
        pltpu.make_async_copy(dst_hbm.at[pl.ds(0, EB)], didx[b], isem[b]).wait()

    def fire_gather(b):
        pltpu.async_copy(x_hbm.at[sidx[b]], rows[b], gsem[b])

    def wait_gather(b):
        pltpu.make_async_copy(zrow_hbm.at[pl.ds(0, EB)], rows[b], gsem[b]).wait()

    def scatter(b):
        pltpu.sync_copy(rows[b], agg_sh.at[didx[b]], add=True)

    fire_idx(0, 0)
    wait_idx(0)
    fire_gather(0)
    fire_idx(1, 1)

    def pair(i, carry):
        for b in (0, 1):
            j = 2 * i + b
            wait_idx(1 - b)
            fire_gather(1 - b)
            wait_gather(b)
            scatter(b)
            fire_idx(j + 2, b)
        return carry

    lax.fori_loop(0, 38, pair, 0)
    wait_idx(1)
    fire_gather(1)
    wait_gather(0)
    scatter(0)
    wait_gather(1)
    scatter(1)

    @pl.when(w < 4)
    def _():
        e0 = (78 * NW + w) * EB
        pltpu.sync_copy(src_hbm.at[pl.ds(e0, EB)], sidx0)
        pltpu.sync_copy(dst_hbm.at[pl.ds(e0, EB)], didx0)
        pltpu.async_copy(x_hbm.at[sidx0], rows0, gsem0).wait()
        pltpu.sync_copy(rows0, agg_sh.at[didx0], add=True)

    plsc.subcore_barrier()
    _writeback_slice(agg_sh, agg_out, c, s)


def _sc_count_body(dst_hbm, zrow_hbm, ones_hbm,
                   cnt_out,
                   cnt_sh, didx0, didx1, ones, isem0, isem1):
    c = lax.axis_index("c")
    s = lax.axis_index("s")
    w = c * NS + s

    _zero_slice(zrow_hbm, cnt_sh, s)
    pltpu.sync_copy(ones_hbm.at[pl.ds(0, EB)], ones)
    plsc.subcore_barrier()

    didx = (didx0, didx1)
    isem = (isem0, isem1)
    g0 = 78 * w

    def fire_idx(j, b):
        pltpu.async_copy(dst_hbm.at[pl.ds((g0 + j) * EB, EB)], didx[b], isem[b])

    def wait_idx(b):
        pltpu.make_async_copy(dst_hbm.at[pl.ds(0, EB)], didx[b], isem[b]).wait()

    fire_idx(0, 0)
    fire_idx(1, 1)

    def pair(i, carry):
        for b in (0, 1):
            j = 2 * i + b
            wait_idx(b)
            pltpu.sync_copy(ones, cnt_sh.at[didx[b]], add=True)
            fire_idx(j + 2, b)
        return carry

    lax.fori_loop(0, 38, pair, 0)
    wait_idx(0)
    pltpu.sync_copy(ones, cnt_sh.at[didx0], add=True)
    wait_idx(1)
    pltpu.sync_copy(ones, cnt_sh.at[didx1], add=True)

    @pl.when(w < 4)
    def _():
        e0 = (78 * NW + w) * EB
        pltpu.sync_copy(dst_hbm.at[pl.ds(e0, EB)], didx0)
        pltpu.sync_copy(ones, cnt_sh.at[didx0], add=True)

    plsc.subcore_barrier()
    _writeback_slice(cnt_sh, cnt_out, c, s)


def _sc_conv2_body(h_hbm, src_hbm, dst_hbm, zrow_hbm,
                   agg_out,
                   agg_sh, sidx0, sidx1, didx0, didx1,
                   rows0, rows1, gsem0, gsem1, isem0, isem1):
    c = lax.axis_index("c")
    s = lax.axis_index("s")

    _zero_slice(zrow_hbm, agg_sh, s)
    plsc.subcore_barrier()

    rows = (rows0, rows1)
    gsem = (gsem0, gsem1)
    sidx = (sidx0, sidx1)
    didx = (didx0, didx1)
    isem = (isem0, isem1)
    base16 = jnp.full((16,), 0, jnp.int32) + c * N_NODES
    g0 = 156 * s

    def add_base(b):
        for k in range(EB // 16):
            sl = pl.ds(k * 16, 16)
            sidx[b][sl] = sidx[b][sl] + base16

    def fire_idx(j, b):
        e0 = (g0 + j) * EB
        pltpu.async_copy(src_hbm.at[pl.ds(e0, EB)], sidx[b], isem[b])
        pltpu.async_copy(dst_hbm.at[pl.ds(e0, EB)], didx[b], isem[b])

    def wait_idx(b):
        pltpu.make_async_copy(src_hbm.at[pl.ds(0, EB)], sidx[b], isem[b]).wait()
        pltpu.make_async_copy(dst_hbm.at[pl.ds(0, EB)], didx[b], isem[b]).wait()

    def fire_gather(b):
        pltpu.async_copy(h_hbm.at[sidx[b]], rows[b], gsem[b])

    def wait_gather(b):
        pltpu.make_async_copy(zrow_hbm.at[pl.ds(0, EB)], rows[b], gsem[b]).wait()

    def scatter(b):
        pltpu.sync_copy(rows[b], agg_sh.at[didx[b]], add=True)

    fire_idx(0, 0)
    wait_idx(0)
    add_base(0)
    fire_gather(0)
    fire_idx(1, 1)

    def pair(i, carry):
        for b in (0, 1):
            j = 2 * i + b
            wait_idx(1 - b)
            add_base(1 - b)
            fire_gather(1 - b)
            wait_gather(b)
            scatter(b)
            fire_idx(j + 2, b)
        return carry

    lax.fori_loop(0, 77, pair, 0)
    wait_idx(1)
    add_base(1)
    fire_gather(1)
    wait_gather(0)
    scatter(0)
    wait_gather(1)
    scatter(1)

    @pl.when(s < 4)
    def _():
        e0 = (156 * NS + s) * EB
        pltpu.sync_copy(src_hbm.at[pl.ds(e0, EB)], sidx0)
        pltpu.sync_copy(dst_hbm.at[pl.ds(e0, EB)], didx0)
        add_base(0)
        pltpu.async_copy(h_hbm.at[sidx0], rows0, gsem0).wait()
        pltpu.sync_copy(rows0, agg_sh.at[didx0], add=True)

    plsc.subcore_barrier()
    _writeback_slice(agg_sh, agg_out, c, s)


_DN = (((1,), (1,)), ((), ()))


def _dot(a, b):
    return lax.dot_general(a, b, _DN, preferred_element_type=jnp.float32)


def _tc1_body(x_ref, p_ref, cntt_ref, wl1_ref, bl1_ref, wr1_ref, g1_ref,
              b1_ref, w1_ref, bw1_ref, w2_ref, bw2_ref, hs_ref):
    cnt = cntt_ref[0][:, 0:1] + cntt_ref[1][:, 0:1]
    mean = (p_ref[0] + p_ref[1]) / jnp.clip(cnt, 1.0)
    pre = _dot(mean, wl1_ref[...]) + bl1_ref[...] + _dot(x_ref[...], wr1_ref[...])
    mu = jnp.mean(pre, axis=-1, keepdims=True)
    var = jnp.mean((pre - mu) ** 2, axis=-1, keepdims=True)
    h = jnp.maximum((pre - mu) * lax.rsqrt(var + EPS) * g1_ref[...] + b1_ref[...], 0.0)
    t = jnp.maximum(_dot(h, w1_ref[...]) + bw1_ref[...], 0.0)
    h2 = jnp.maximum(_dot(t, w2_ref[...]) + bw2_ref[...], 0.0)
    hs_ref[0] = h2[:, :128]
    hs_ref[1] = h2[:, 128:]


def _tc2_body(hs_ref, q_ref, cntt_ref, wl2_ref, bl2_ref, wr2_ref, g2_ref,
              b2_ref, w3_ref, bw3_ref, w4_ref, bw4_ref, o_ref):
    cnt = cntt_ref[0][:, 0:1] + cntt_ref[1][:, 0:1]
    icnt = 1.0 / jnp.clip(cnt, 1.0)
    mean = jnp.concatenate([q_ref[0], q_ref[1]], axis=1) * icnt
    hfull = jnp.concatenate([hs_ref[0], hs_ref[1]], axis=1)
    pre = _dot(mean, wl2_ref[...]) + bl2_ref[...] + _dot(hfull, wr2_ref[...])
    mu = jnp.mean(pre, axis=-1, keepdims=True)
    var = jnp.mean((pre - mu) ** 2, axis=-1, keepdims=True)
    h = jnp.maximum((pre - mu) * lax.rsqrt(var + EPS) * g2_ref[...] + b2_ref[...], 0.0)
    t = jnp.maximum(_dot(h, w3_ref[...]) + bw3_ref[...], 0.0)
    o_ref[...] = _dot(t, w4_ref[...]) + bw4_ref[...]


def _full(shape):
    return pl.BlockSpec(shape, lambda i: tuple(0 for _ in shape))


_AGG_SCRATCH = [
    pltpu.VMEM_SHARED((N_NODES, 128), jnp.float32),
    pltpu.VMEM((EB,), jnp.int32),
    pltpu.VMEM((EB,), jnp.int32),
    pltpu.VMEM((EB,), jnp.int32),
    pltpu.VMEM((EB,), jnp.int32),
    pltpu.VMEM((EB, 128), jnp.float32),
    pltpu.VMEM((EB, 128), jnp.float32),
    pltpu.SemaphoreType.DMA,
    pltpu.SemaphoreType.DMA,
    pltpu.SemaphoreType.DMA,
    pltpu.SemaphoreType.DMA,
]


def _sc_conv1(x, src, dst, zrow):
    return pl.kernel(
        _sc_conv1_body,
        out_type=jax.ShapeDtypeStruct((NC, N_NODES, 128), jnp.float32),
        mesh=plsc.VectorSubcoreMesh(**_SC_MESH),
        scratch_types=list(_AGG_SCRATCH),
    )(x, src, dst, zrow)


def _sc_count(dst, zrow, ones_in):
    return pl.kernel(
        _sc_count_body,
        out_type=jax.ShapeDtypeStruct((NC, N_NODES, 128), jnp.float32),
        mesh=plsc.VectorSubcoreMesh(**_SC_MESH),
        scratch_types=[
            pltpu.VMEM_SHARED((N_NODES, 128), jnp.float32),
            pltpu.VMEM((EB,), jnp.int32),
            pltpu.VMEM((EB,), jnp.int32),
            pltpu.VMEM((EB, 128), jnp.float32),
            pltpu.SemaphoreType.DMA,
            pltpu.SemaphoreType.DMA,
        ],
    )(dst, zrow, ones_in)


def _sc_conv2(hflat, src, dst, zrow):
    return pl.kernel(
        _sc_conv2_body,
        out_type=jax.ShapeDtypeStruct((NC, N_NODES, 128), jnp.float32),
        mesh=plsc.VectorSubcoreMesh(**_SC_MESH),
        scratch_types=list(_AGG_SCRATCH),
    )(hflat, src, dst, zrow)


def kernel(x, edge_index, Wl1, bl1, Wr1, g1, beta1, W1, bW1, W2, bW2,
           Wl2, bl2, Wr2, g2, beta2, W3, bW3, W4, bW4):
    ei = edge_index.astype(jnp.int32)
    src, dst = ei[0], ei[1]
    zrow = jnp.zeros((RPT, 128), jnp.float32)
    ones_in = jnp.ones((EB, 128), jnp.float32)

    agg1 = _sc_conv1(x, src, dst, zrow)
    cntt = _sc_count(dst, zrow, ones_in)

    R = 1000
    grid = (N_NODES // R,)
    b1 = bl1.reshape(1, -1)
    gg1 = g1.reshape(1, -1)
    bb1 = beta1.reshape(1, -1)
    hs = pl.pallas_call(
        _tc1_body,
        grid=grid,
        in_specs=[
            pl.BlockSpec((R, 128), lambda i: (i, 0)),
            pl.BlockSpec((NC, R, 128), lambda i: (0, i, 0)),
            pl.BlockSpec((NC, R, 128), lambda i: (0, i, 0)),
            _full(Wl1.shape), _full(b1.shape), _full(Wr1.shape),
            _full(gg1.shape), _full(bb1.shape),
            _full(W1.shape), _full((1, 512)),
            _full(W2.shape), _full((1, 256)),
        ],
        out_specs=pl.BlockSpec((NC, R, 128), lambda i: (0, i, 0)),
        out_shape=jax.ShapeDtypeStruct((NC, N_NODES, 128), jnp.float32),
    )(x, agg1, cntt, Wl1, b1, Wr1, gg1, bb1,
      W1, bW1.reshape(1, -1), W2, bW2.reshape(1, -1))

    hflat = hs.reshape(NC * N_NODES, 128)
    agg2 = _sc_conv2(hflat, src, dst, zrow)

    out = pl.pallas_call(
        _tc2_body,
        grid=grid,
        in_specs=[
            pl.BlockSpec((NC, R, 128), lambda i: (0, i, 0)),
            pl.BlockSpec((NC, R, 128), lambda i: (0, i, 0)),
            pl.BlockSpec((NC, R, 128), lambda i: (0, i, 0)),
            _full(Wl2.shape), _full((1, 128)), _full(Wr2.shape),
            _full((1, 128)), _full((1, 128)),
            _full(W3.shape), _full((1, 256)),
            _full(W4.shape), _full((1, 128)),
        ],
        out_specs=pl.BlockSpec((R, 128), lambda i: (i, 0)),
        out_shape=jax.ShapeDtypeStruct((N_NODES, 128), jnp.float32),
    )(hs, agg2, cntt, Wl2, bl2.reshape(1, -1), Wr2,
      g2.reshape(1, -1), beta2.reshape(1, -1),
      W3, bW3.reshape(1, -1), W4, bW4.reshape(1, -1))

    return out


# speedup vs baseline: 9.0340x; 1.0166x over previous
import functools

import jax
import jax.numpy as jnp
from jax import lax
from jax.experimental import pallas as pl
from jax.experimental.pallas import tpu as pltpu
from jax.experimental.pallas import tpu_sc as plsc

N_NODES = 10000
N_EDGES = 320000
NC = 2
NS = 16
NW = NC * NS
EB = 128
NB_TOTAL = N_EDGES // EB
EPS = 1e-5

RPT = 624

_SC_MESH = dict(core_axis_name="c", subcore_axis_name="s",
                num_cores=NC, num_subcores=NS)


def _zero_slice(zrow_hbm, sh, s):
    r0 = s * RPT
    pltpu.sync_copy(zrow_hbm.at[pl.ds(0, RPT)], sh.at[pl.ds(r0, RPT)])

    @pl.when(s == NS - 1)
    def _():
        pltpu.sync_copy(zrow_hbm.at[pl.ds(0, 16)], sh.at[pl.ds(NS * RPT, 16)])


def _writeback_slice(sh, out, c, s):
    r0 = s * RPT
    pltpu.sync_copy(sh.at[pl.ds(r0, RPT)], out.at[c, pl.ds(r0, RPT)])

    @pl.when(s == NS - 1)
    def _():
        pltpu.sync_copy(sh.at[pl.ds(NS * RPT, 16)],
                        out.at[c, pl.ds(NS * RPT, 16)])


def _sc_conv1_body(x_hbm, src_hbm, dst_hbm, zrow_hbm, ones_hbm,
                   agg_out, cnt_out,
                   agg_sh, sidx0, sidx1, didx0, didx1,
                   rows0, rows1, gsem0, gsem1, isem0, isem1):
    c = lax.axis_index("c")
    s = lax.axis_index("s")
    w = c * NS + s

    _zero_slice(zrow_hbm, agg_sh, s)
    plsc.subcore_barrier()

    rows = (rows0, rows1)
    gsem = (gsem0, gsem1)
    sidx = (sidx0, sidx1)
    didx = (didx0, didx1)
    isem = (isem0, isem1)
    g0 = 78 * w

    def fire_idx(j, b):
        e0 = (g0 + j) * EB
        pltpu.async_copy(src_hbm.at[pl.ds(e0, EB)], sidx[b], isem[b])
        pltpu.async_copy(dst_hbm.at[pl.ds(e0, EB)], didx[b], isem[b])

    def wait_idx(b):
        pltpu.make_async_copy(src_hbm.at[pl.ds(0, EB)], sidx[b], isem[b]).wait()
        pltpu.make_async_copy(dst_hbm.at[pl.ds(0, EB)], didx[b], isem[b]).wait()

    def fire_gather(b):
        pltpu.async_copy(x_hbm.at[sidx[b]], rows[b], gsem[b])

    def wait_gather(b):
        pltpu.make_async_copy(zrow_hbm.at[pl.ds(0, EB)], rows[b], gsem[b]).wait()

    def scatter(b):
        pltpu.sync_copy(rows[b], agg_sh.at[didx[b]], add=True)

    fire_idx(0, 0)
    wait_idx(0)
    fire_gather(0)
    fire_idx(1, 1)

    def pair(i, carry):
        for b in (0, 1):
            j = 2 * i + b
            wait_idx(1 - b)
            fire_gather(1 - b)
            wait_gather(b)
            scatter(b)
            fire_idx(j + 2, b)
        return carry

    lax.fori_loop(0, 38, pair, 0)
    wait_idx(1)
    fire_gather(1)
    wait_gather(0)
    scatter(0)
    wait_gather(1)
    scatter(1)

    @pl.when(w < 4)
    def _():
        e0 = (78 * NW + w) * EB
        pltpu.sync_copy(src_hbm.at[pl.ds(e0, EB)], sidx0)
        pltpu.sync_copy(dst_hbm.at[pl.ds(e0, EB)], didx0)
        pltpu.async_copy(x_hbm.at[sidx0], rows0, gsem0).wait()
        pltpu.sync_copy(rows0, agg_sh.at[didx0], add=True)

    plsc.subcore_barrier()
    _writeback_slice(agg_sh, agg_out, c, s)

    _zero_slice(zrow_hbm, agg_sh, s)
    pltpu.sync_copy(ones_hbm.at[pl.ds(0, EB)], rows0)
    plsc.subcore_barrier()

    def fire_didx(j, b):
        pltpu.async_copy(dst_hbm.at[pl.ds((g0 + j) * EB, EB)], didx[b], isem[b])

    def wait_didx(b):
        pltpu.make_async_copy(dst_hbm.at[pl.ds(0, EB)], didx[b], isem[b]).wait()

    fire_didx(0, 0)
    fire_didx(1, 1)

    def cpair(i, carry):
        for b in (0, 1):
            j = 2 * i + b
            wait_didx(b)
            pltpu.sync_copy(rows0, agg_sh.at[didx[b]], add=True)
            fire_didx(j + 2, b)
        return carry

    lax.fori_loop(0, 38, cpair, 0)
    wait_didx(0)
    pltpu.sync_copy(rows0, agg_sh.at[didx0], add=True)
    wait_didx(1)
    pltpu.sync_copy(rows0, agg_sh.at[didx1], add=True)

    @pl.when(w < 4)
    def _():
        e0 = (78 * NW + w) * EB
        pltpu.sync_copy(dst_hbm.at[pl.ds(e0, EB)], didx0)
        pltpu.sync_copy(rows0, agg_sh.at[didx0], add=True)

    plsc.subcore_barrier()
    _writeback_slice(agg_sh, cnt_out, c, s)


def _sc_conv2_body(h_hbm, src_hbm, dst_hbm, zrow_hbm,
                   agg_out,
                   agg_sh, sidx0, sidx1, didx0, didx1,
                   rows0, rows1, gsem0, gsem1, isem0, isem1):
    c = lax.axis_index("c")
    s = lax.axis_index("s")

    _zero_slice(zrow_hbm, agg_sh, s)
    plsc.subcore_barrier()

    rows = (rows0, rows1)
    gsem = (gsem0, gsem1)
    sidx = (sidx0, sidx1)
    didx = (didx0, didx1)
    isem = (isem0, isem1)
    base16 = jnp.full((16,), 0, jnp.int32) + c * N_NODES
    g0 = 156 * s

    def add_base(b):
        for k in range(EB // 16):
            sl = pl.ds(k * 16, 16)
            sidx[b][sl] = sidx[b][sl] + base16

    def fire_idx(j, b):
        e0 = (g0 + j) * EB
        pltpu.async_copy(src_hbm.at[pl.ds(e0, EB)], sidx[b], isem[b])
        pltpu.async_copy(dst_hbm.at[pl.ds(e0, EB)], didx[b], isem[b])

    def wait_idx(b):
        pltpu.make_async_copy(src_hbm.at[pl.ds(0, EB)], sidx[b], isem[b]).wait()
        pltpu.make_async_copy(dst_hbm.at[pl.ds(0, EB)], didx[b], isem[b]).wait()

    def fire_gather(b):
        pltpu.async_copy(h_hbm.at[sidx[b]], rows[b], gsem[b])

    def wait_gather(b):
        pltpu.make_async_copy(zrow_hbm.at[pl.ds(0, EB)], rows[b], gsem[b]).wait()

    def scatter(b):
        pltpu.sync_copy(rows[b], agg_sh.at[didx[b]], add=True)

    fire_idx(0, 0)
    wait_idx(0)
    add_base(0)
    fire_gather(0)
    fire_idx(1, 1)

    def pair(i, carry):
        for b in (0, 1):
            j = 2 * i + b
            wait_idx(1 - b)
            add_base(1 - b)
            fire_gather(1 - b)
            wait_gather(b)
            scatter(b)
            fire_idx(j + 2, b)
        return carry

    lax.fori_loop(0, 77, pair, 0)
    wait_idx(1)
    add_base(1)
    fire_gather(1)
    wait_gather(0)
    scatter(0)
    wait_gather(1)
    scatter(1)

    @pl.when(s < 4)
    def _():
        e0 = (156 * NS + s) * EB
        pltpu.sync_copy(src_hbm.at[pl.ds(e0, EB)], sidx0)
        pltpu.sync_copy(dst_hbm.at[pl.ds(e0, EB)], didx0)
        add_base(0)
        pltpu.async_copy(h_hbm.at[sidx0], rows0, gsem0).wait()
        pltpu.sync_copy(rows0, agg_sh.at[didx0], add=True)

    plsc.subcore_barrier()
    _writeback_slice(agg_sh, agg_out, c, s)


_DN = (((1,), (1,)), ((), ()))


def _dot(a, b):
    return lax.dot_general(a, b, _DN, preferred_element_type=jnp.float32)


def _tc1_body(x_ref, p_ref, cntt_ref, wl1_ref, bl1_ref, wr1_ref, g1_ref,
              b1_ref, w1_ref, bw1_ref, w2_ref, bw2_ref, hs_ref):
    cnt = cntt_ref[0][:, 0:1] + cntt_ref[1][:, 0:1]
    mean = (p_ref[0] + p_ref[1]) / jnp.clip(cnt, 1.0)
    pre = _dot(mean, wl1_ref[...]) + bl1_ref[...] + _dot(x_ref[...], wr1_ref[...])
    mu = jnp.mean(pre, axis=-1, keepdims=True)
    var = jnp.mean((pre - mu) ** 2, axis=-1, keepdims=True)
    h = jnp.maximum((pre - mu) * lax.rsqrt(var + EPS) * g1_ref[...] + b1_ref[...], 0.0)
    t = jnp.maximum(_dot(h, w1_ref[...]) + bw1_ref[...], 0.0)
    h2 = jnp.maximum(_dot(t, w2_ref[...]) + bw2_ref[...], 0.0)
    hs_ref[0] = h2[:, :128]
    hs_ref[1] = h2[:, 128:]


def _tc2_body(hs_ref, q_ref, cntt_ref, wl2_ref, bl2_ref, wr2_ref, g2_ref,
              b2_ref, w3_ref, bw3_ref, w4_ref, bw4_ref, o_ref):
    cnt = cntt_ref[0][:, 0:1] + cntt_ref[1][:, 0:1]
    icnt = 1.0 / jnp.clip(cnt, 1.0)
    mean = jnp.concatenate([q_ref[0], q_ref[1]], axis=1) * icnt
    hfull = jnp.concatenate([hs_ref[0], hs_ref[1]], axis=1)
    pre = _dot(mean, wl2_ref[...]) + bl2_ref[...] + _dot(hfull, wr2_ref[...])
    mu = jnp.mean(pre, axis=-1, keepdims=True)
    var = jnp.mean((pre - mu) ** 2, axis=-1, keepdims=True)
    h = jnp.maximum((pre - mu) * lax.rsqrt(var + EPS) * g2_ref[...] + b2_ref[...], 0.0)
    t = jnp.maximum(_dot(h, w3_ref[...]) + bw3_ref[...], 0.0)
    o_ref[...] = _dot(t, w4_ref[...]) + bw4_ref[...]


def _full(shape):
    return pl.BlockSpec(shape, lambda i: tuple(0 for _ in shape))


_AGG_SCRATCH = [
    pltpu.VMEM_SHARED((N_NODES, 128), jnp.float32),
    pltpu.VMEM((EB,), jnp.int32),
    pltpu.VMEM((EB,), jnp.int32),
    pltpu.VMEM((EB,), jnp.int32),
    pltpu.VMEM((EB,), jnp.int32),
    pltpu.VMEM((EB, 128), jnp.float32),
    pltpu.VMEM((EB, 128), jnp.float32),
    pltpu.SemaphoreType.DMA,
    pltpu.SemaphoreType.DMA,
    pltpu.SemaphoreType.DMA,
    pltpu.SemaphoreType.DMA,
]


def _sc_conv1(x, src, dst, zrow, ones_in):
    return pl.kernel(
        _sc_conv1_body,
        out_type=(jax.ShapeDtypeStruct((NC, N_NODES, 128), jnp.float32),
                  jax.ShapeDtypeStruct((NC, N_NODES, 128), jnp.float32)),
        mesh=plsc.VectorSubcoreMesh(**_SC_MESH),
        scratch_types=list(_AGG_SCRATCH),
    )(x, src, dst, zrow, ones_in)


def _sc_conv2(hflat, src, dst, zrow):
    return pl.kernel(
        _sc_conv2_body,
        out_type=jax.ShapeDtypeStruct((NC, N_NODES, 128), jnp.float32),
        mesh=plsc.VectorSubcoreMesh(**_SC_MESH),
        scratch_types=list(_AGG_SCRATCH),
    )(hflat, src, dst, zrow)


def kernel(x, edge_index, Wl1, bl1, Wr1, g1, beta1, W1, bW1, W2, bW2,
           Wl2, bl2, Wr2, g2, beta2, W3, bW3, W4, bW4):
    ei = edge_index.astype(jnp.int32)
    src, dst = ei[0], ei[1]
    zrow = jnp.zeros((RPT, 128), jnp.float32)
    ones_in = jnp.ones((EB, 128), jnp.float32)

    agg1, cntt = _sc_conv1(x, src, dst, zrow, ones_in)

    R = 1000
    grid = (N_NODES // R,)
    b1 = bl1.reshape(1, -1)
    gg1 = g1.reshape(1, -1)
    bb1 = beta1.reshape(1, -1)
    hs = pl.pallas_call(
        _tc1_body,
        grid=grid,
        in_specs=[
            pl.BlockSpec((R, 128), lambda i: (i, 0)),
            pl.BlockSpec((NC, R, 128), lambda i: (0, i, 0)),
            pl.BlockSpec((NC, R, 128), lambda i: (0, i, 0)),
            _full(Wl1.shape), _full(b1.shape), _full(Wr1.shape),
            _full(gg1.shape), _full(bb1.shape),
            _full(W1.shape), _full((1, 512)),
            _full(W2.shape), _full((1, 256)),
        ],
        out_specs=pl.BlockSpec((NC, R, 128), lambda i: (0, i, 0)),
        out_shape=jax.ShapeDtypeStruct((NC, N_NODES, 128), jnp.float32),
    )(x, agg1, cntt, Wl1, b1, Wr1, gg1, bb1,
      W1, bW1.reshape(1, -1), W2, bW2.reshape(1, -1))

    hflat = hs.reshape(NC * N_NODES, 128)
    agg2 = _sc_conv2(hflat, src, dst, zrow)

    out = pl.pallas_call(
        _tc2_body,
        grid=grid,
        in_specs=[
            pl.BlockSpec((NC, R, 128), lambda i: (0, i, 0)),
            pl.BlockSpec((NC, R, 128), lambda i: (0, i, 0)),
            pl.BlockSpec((NC, R, 128), lambda i: (0, i, 0)),
            _full(Wl2.shape), _full((1, 128)), _full(Wr2.shape),
            _full((1, 128)), _full((1, 128)),
            _full(W3.shape), _full((1, 256)),
            _full(W4.shape), _full((1, 128)),
        ],
        out_specs=pl.BlockSpec((R, 128), lambda i: (i, 0)),
        out_shape=jax.ShapeDtypeStruct((N_NODES, 128), jnp.float32),
    )(hs, agg2, cntt, Wl2, bl2.reshape(1, -1), Wr2,
      g2.reshape(1, -1), beta2.reshape(1, -1),
      W3, bW3.reshape(1, -1), W4, bW4.reshape(1, -1))

    return out
